# baseline (speedup 1.0000x reference)
import functools

import jax
import jax.numpy as jnp
from jax import lax
from jax.experimental import pallas as pl
from jax.experimental.pallas import tpu as pltpu
from jax.experimental.pallas import tpu_sc as plsc

_RB = 512
_EHC = 1024


def _argmin_body(rb, zb_ref, eb_ref, ef_ref, idx_ref, eh_sc):
    s = pl.program_id(0)
    kk, d = ef_ref.shape
    nchunk = kk // 128

    @pl.when(s == 0)
    def _():
        for c0 in range(0, kk, _EHC):
            ef = ef_ref[c0:c0 + _EHC, :]
            e_sq = ef * ef
            eh_sc[0:1, c0:c0 + _EHC] = 0.5 * lax.dot_general(
                jnp.ones((8, d), jnp.float32), e_sq, (((1,), (1,)), ((), ())),
                preferred_element_type=jnp.float32,
                precision=lax.Precision.HIGHEST,
            )[0:1, :]

    row = pl.ds(s * rb, rb)
    zs = zb_ref[row, :]

    rm = jnp.full((rb, 128), 3.0e38, jnp.float32)
    ri = jnp.zeros((rb, 128), jnp.int32)
    for c in range(0, nchunk, 2):
        sc2 = []
        for h in range(2):
            col0 = (c + h) * 128
            cross = lax.dot_general(
                zs, eb_ref[col0:col0 + 128, :], (((1,), (1,)), ((), ())),
                preferred_element_type=jnp.float32,
            )
            sc2.append(eh_sc[0:1, col0:col0 + 128] - cross)
        mp = sc2[1] < sc2[0]
        scm = jnp.minimum(sc2[0], sc2[1])
        idm = jnp.where(mp, jnp.int32(c + 1), jnp.int32(c))
        m = scm < rm
        rm = jnp.where(m, scm, rm)
        ri = jnp.where(m, idm, ri)

    lane = lax.broadcasted_iota(jnp.int32, (rb, 128), 1)
    gidx = ri * 128 + lane
    row_min = jnp.min(rm, axis=1, keepdims=True)
    cand = jnp.where(rm == row_min, gidx, jnp.int32(2**30))
    idx_ref[row, :] = jnp.min(cand, axis=1, keepdims=True)


def _nearest_idx(z, embeddings):
    b, d = z.shape
    kk, _ = embeddings.shape
    rb = _RB
    assert b % rb == 0 and kk % 128 == 0
    return pl.pallas_call(
        functools.partial(_argmin_body, rb),
        grid=(b // rb,),
        in_specs=[
            pl.BlockSpec((b, d), lambda s: (0, 0)),
            pl.BlockSpec((kk, d), lambda s: (0, 0)),
            pl.BlockSpec((kk, d), lambda s: (0, 0)),
        ],
        out_specs=pl.BlockSpec((b, 1), lambda s: (0, 0)),
        out_shape=jax.ShapeDtypeStruct((b, 1), jnp.int32),
        scratch_shapes=[
            pltpu.VMEM((1, kk), jnp.float32),
        ],
        compiler_params=pltpu.CompilerParams(
            dimension_semantics=("arbitrary",),
        ),
    )(z.astype(jnp.bfloat16), embeddings.astype(jnp.bfloat16), embeddings)


def _sc_gather(table, idx):
    kk, d = table.shape
    b = idx.shape[0]
    info = plsc.get_sparse_core_info()
    nw = info.num_cores * info.num_subcores
    assert b % (8 * nw) == 0
    b_per_w = b // nw
    mesh = plsc.VectorSubcoreMesh(core_axis_name="c", subcore_axis_name="s")

    @functools.partial(
        pl.kernel,
        mesh=mesh,
        out_type=jax.ShapeDtypeStruct((b, d), jnp.float32),
        scratch_types=[
            pltpu.VMEM((b_per_w,), jnp.int32),
            pltpu.VMEM((b_per_w, d), jnp.float32),
            pltpu.SemaphoreType.DMA,
        ],
    )
    def gather_kernel(table_hbm, idx_hbm, out_hbm, idx_v, rows_v, sem):
        wid = lax.axis_index("s") * info.num_cores + lax.axis_index("c")
        base = wid * b_per_w
        pltpu.sync_copy(idx_hbm.at[pl.ds(base, b_per_w)], idx_v)
        pltpu.async_copy(table_hbm.at[idx_v], rows_v, sem).wait()
        pltpu.sync_copy(rows_v, out_hbm.at[pl.ds(base, b_per_w)])

    return gather_kernel(table, idx)


def kernel(z, embeddings, batch_size):
    idx = _nearest_idx(z, embeddings).reshape(-1)
    return _sc_gather(embeddings, idx)


# speedup vs baseline: 1.2114x; 1.2114x over previous
import functools

import jax
import jax.numpy as jnp
from jax import lax
from jax.experimental import pallas as pl
from jax.experimental.pallas import tpu as pltpu
from jax.experimental.pallas import tpu_sc as plsc

_RB = 512
_EHC = 1024


def _argmin_body(rb, zb_ref, eb_ref, ef_ref, idx_ref, eh_sc):
    s = pl.program_id(0)
    kk, d = ef_ref.shape
    nchunk = kk // 128

    @pl.when(s == 0)
    def _():
        for c0 in range(0, kk, _EHC):
            ef = ef_ref[c0:c0 + _EHC, :]
            e_sq = ef * ef
            eh_sc[0:1, c0:c0 + _EHC] = 0.5 * lax.dot_general(
                jnp.ones((8, d), jnp.float32), e_sq, (((1,), (1,)), ((), ())),
                preferred_element_type=jnp.float32,
                precision=lax.Precision.HIGHEST,
            )[0:1, :]

    row = pl.ds(s * rb, rb)
    zs = zb_ref[row, :]

    rm = jnp.full((rb, 128), 3.0e38, jnp.float32)
    ri = jnp.zeros((rb, 128), jnp.int32)
    for c in range(0, nchunk, 2):
        col0 = c * 128
        cross = lax.dot_general(
            zs, eb_ref[col0:col0 + 256, :], (((1,), (1,)), ((), ())),
            preferred_element_type=jnp.float32,
        )
        sc_a = eh_sc[0:1, col0:col0 + 128] - cross[:, 0:128]
        sc_b = eh_sc[0:1, col0 + 128:col0 + 256] - cross[:, 128:256]
        mp = sc_b < sc_a
        scm = jnp.minimum(sc_a, sc_b)
        idm = jnp.where(mp, jnp.int32(c + 1), jnp.int32(c))
        m = scm < rm
        rm = jnp.where(m, scm, rm)
        ri = jnp.where(m, idm, ri)

    lane = lax.broadcasted_iota(jnp.int32, (rb, 128), 1)
    gidx = ri * 128 + lane
    row_min = jnp.min(rm, axis=1, keepdims=True)
    cand = jnp.where(rm == row_min, gidx, jnp.int32(2**30))
    idx_ref[row, :] = jnp.min(cand, axis=1, keepdims=True)


def _nearest_idx(z, embeddings):
    b, d = z.shape
    kk, _ = embeddings.shape
    rb = _RB
    assert b % rb == 0 and kk % 128 == 0
    return pl.pallas_call(
        functools.partial(_argmin_body, rb),
        grid=(b // rb,),
        in_specs=[
            pl.BlockSpec((b, d), lambda s: (0, 0)),
            pl.BlockSpec((kk, d), lambda s: (0, 0)),
            pl.BlockSpec((kk, d), lambda s: (0, 0)),
        ],
        out_specs=pl.BlockSpec((b, 1), lambda s: (0, 0)),
        out_shape=jax.ShapeDtypeStruct((b, 1), jnp.int32),
        scratch_shapes=[
            pltpu.VMEM((1, kk), jnp.float32),
        ],
        compiler_params=pltpu.CompilerParams(
            dimension_semantics=("arbitrary",),
        ),
    )(z.astype(jnp.bfloat16), embeddings.astype(jnp.bfloat16), embeddings)


def _sc_gather(table, idx):
    kk, d = table.shape
    b = idx.shape[0]
    info = plsc.get_sparse_core_info()
    nw = info.num_cores * info.num_subcores
    assert b % (8 * nw) == 0
    b_per_w = b // nw
    mesh = plsc.VectorSubcoreMesh(core_axis_name="c", subcore_axis_name="s")

    @functools.partial(
        pl.kernel,
        mesh=mesh,
        out_type=jax.ShapeDtypeStruct((b, d), jnp.float32),
        scratch_types=[
            pltpu.VMEM((b_per_w,), jnp.int32),
            pltpu.VMEM((b_per_w, d), jnp.float32),
            pltpu.SemaphoreType.DMA,
        ],
    )
    def gather_kernel(table_hbm, idx_hbm, out_hbm, idx_v, rows_v, sem):
        wid = lax.axis_index("s") * info.num_cores + lax.axis_index("c")
        base = wid * b_per_w
        pltpu.sync_copy(idx_hbm.at[pl.ds(base, b_per_w)], idx_v)
        pltpu.async_copy(table_hbm.at[idx_v], rows_v, sem).wait()
        pltpu.sync_copy(rows_v, out_hbm.at[pl.ds(base, b_per_w)])

    return gather_kernel(table, idx)


def kernel(z, embeddings, batch_size):
    idx = _nearest_idx(z, embeddings).reshape(-1)
    return _sc_gather(embeddings, idx)


# speedup vs baseline: 1.2348x; 1.0193x over previous
import functools

import jax
import jax.numpy as jnp
from jax import lax
from jax.experimental import pallas as pl
from jax.experimental.pallas import tpu as pltpu
from jax.experimental.pallas import tpu_sc as plsc

_RB = 1024
_EHC = 1024


def _argmin_body(rb, zb_ref, eb_ref, ef_ref, idx_ref, eh_sc):
    s = pl.program_id(0)
    kk, d = ef_ref.shape
    nchunk = kk // 128

    @pl.when(s == 0)
    def _():
        for c0 in range(0, kk, _EHC):
            ef = ef_ref[c0:c0 + _EHC, :]
            e_sq = ef * ef
            eh_sc[0:1, c0:c0 + _EHC] = 0.5 * lax.dot_general(
                jnp.ones((8, d), jnp.float32), e_sq, (((1,), (1,)), ((), ())),
                preferred_element_type=jnp.float32,
                precision=lax.Precision.HIGHEST,
            )[0:1, :]

    row = pl.ds(s * rb, rb)
    zs = zb_ref[row, :]

    rm = jnp.full((rb, 128), 3.0e38, jnp.float32)
    ri = jnp.zeros((rb, 128), jnp.int32)
    for c in range(0, nchunk, 2):
        col0 = c * 128
        cross = lax.dot_general(
            zs, eb_ref[col0:col0 + 256, :], (((1,), (1,)), ((), ())),
            preferred_element_type=jnp.float32,
        )
        sc_a = eh_sc[0:1, col0:col0 + 128] - cross[:, 0:128]
        sc_b = eh_sc[0:1, col0 + 128:col0 + 256] - cross[:, 128:256]
        mp = sc_b < sc_a
        scm = jnp.minimum(sc_a, sc_b)
        idm = jnp.where(mp, jnp.int32(c + 1), jnp.int32(c))
        m = scm < rm
        rm = jnp.where(m, scm, rm)
        ri = jnp.where(m, idm, ri)

    lane = lax.broadcasted_iota(jnp.int32, (rb, 128), 1)
    gidx = ri * 128 + lane
    row_min = jnp.min(rm, axis=1, keepdims=True)
    cand = jnp.where(rm == row_min, gidx, jnp.int32(2**30))
    idx_ref[row, :] = jnp.min(cand, axis=1, keepdims=True)


def _nearest_idx(z, embeddings):
    b, d = z.shape
    kk, _ = embeddings.shape
    rb = _RB
    assert b % rb == 0 and kk % 128 == 0
    return pl.pallas_call(
        functools.partial(_argmin_body, rb),
        grid=(b // rb,),
        in_specs=[
            pl.BlockSpec((b, d), lambda s: (0, 0)),
            pl.BlockSpec((kk, d), lambda s: (0, 0)),
            pl.BlockSpec((kk, d), lambda s: (0, 0)),
        ],
        out_specs=pl.BlockSpec((b, 1), lambda s: (0, 0)),
        out_shape=jax.ShapeDtypeStruct((b, 1), jnp.int32),
        scratch_shapes=[
            pltpu.VMEM((1, kk), jnp.float32),
        ],
        compiler_params=pltpu.CompilerParams(
            dimension_semantics=("arbitrary",),
        ),
    )(z.astype(jnp.bfloat16), embeddings.astype(jnp.bfloat16), embeddings)


def _sc_gather(table, idx):
    kk, d = table.shape
    b = idx.shape[0]
    info = plsc.get_sparse_core_info()
    nw = info.num_cores * info.num_subcores
    assert b % (8 * nw) == 0
    b_per_w = b // nw
    mesh = plsc.VectorSubcoreMesh(core_axis_name="c", subcore_axis_name="s")

    @functools.partial(
        pl.kernel,
        mesh=mesh,
        out_type=jax.ShapeDtypeStruct((b, d), jnp.float32),
        scratch_types=[
            pltpu.VMEM((b_per_w,), jnp.int32),
            pltpu.VMEM((b_per_w, d), jnp.float32),
            pltpu.SemaphoreType.DMA,
        ],
    )
    def gather_kernel(table_hbm, idx_hbm, out_hbm, idx_v, rows_v, sem):
        wid = lax.axis_index("s") * info.num_cores + lax.axis_index("c")
        base = wid * b_per_w
        pltpu.sync_copy(idx_hbm.at[pl.ds(base, b_per_w)], idx_v)
        pltpu.async_copy(table_hbm.at[idx_v], rows_v, sem).wait()
        pltpu.sync_copy(rows_v, out_hbm.at[pl.ds(base, b_per_w)])

    return gather_kernel(table, idx)


def kernel(z, embeddings, batch_size):
    idx = _nearest_idx(z, embeddings).reshape(-1)
    return _sc_gather(embeddings, idx)
